# baseline (speedup 1.0000x reference)
import functools

import jax
import jax.numpy as jnp
from jax import lax
from jax.experimental import pallas as pl
from jax.experimental.pallas import tpu as pltpu
from jax.experimental.pallas import tpu_sc as plsc

_IGNORE = 255
_THRESH = 0.6
_THRESH_BITS = 0x3F19999A
_INF_BITS = 0x7F800000
_MIN_KEPT = 256
_KEPT_MIN_LOSS = 0.25

_B, _C, _H, _W = 8, 19, 512, 512
_BH = 512
_N = _B * _H * _W
_ROWS = _N // _W


def _softmax_stats(score_ref, tgt_ref):
    t = tgt_ref[0]
    m = score_ref[0, 0]
    for c in range(1, _C):
        m = jnp.maximum(m, score_ref[0, c])
    s = jnp.zeros_like(m)
    xt = jnp.zeros_like(m)
    for c in range(_C):
        xc = score_ref[0, c]
        s = s + jnp.exp(xc - m)
        xt = jnp.where(t == c, xc, xt)
    logz = m + jnp.log(s)
    mask = t != _IGNORE
    return mask, logz, xt


def _pass1_body(score_ref, tgt_ref, ml_ref):
    mask, logz, xt = _softmax_stats(score_ref, tgt_ref)
    loss = logz - xt
    pred = jnp.exp(xt - logz)
    ml_ref[0] = jnp.where(mask & (pred < _THRESH), loss, 0.0)


def _pass1(score, target, b0, nb):
    nh = _H // _BH
    return pl.pallas_call(
        _pass1_body,
        grid=(nb, nh),
        in_specs=[
            pl.BlockSpec((1, _C, _BH, _W), lambda b, h: (b + b0, 0, h, 0)),
            pl.BlockSpec((1, _BH, _W), lambda b, h: (b + b0, h, 0)),
        ],
        out_specs=pl.BlockSpec((1, _BH, _W), lambda b, h: (b, h, 0)),
        out_shape=jax.ShapeDtypeStruct((nb, _H, _W), jnp.float32),
    )(score, target)


def _pass1_full_body(score_ref, tgt_ref, loss_ref, pred_ref):
    mask, logz, xt = _softmax_stats(score_ref, tgt_ref)
    loss_ref[0] = jnp.where(mask, logz - xt, 0.0)
    pred_ref[0] = jnp.where(mask, jnp.exp(xt - logz), jnp.inf)


def _pass1_full(score, target):
    nh = _H // _BH
    return pl.pallas_call(
        _pass1_full_body,
        grid=(_B, nh),
        in_specs=[
            pl.BlockSpec((1, _C, _BH, _W), lambda b, h: (b, 0, h, 0)),
            pl.BlockSpec((1, _BH, _W), lambda b, h: (b, h, 0)),
        ],
        out_specs=[
            pl.BlockSpec((1, _BH, _W), lambda b, h: (b, h, 0)),
            pl.BlockSpec((1, _BH, _W), lambda b, h: (b, h, 0)),
        ],
        out_shape=[
            jax.ShapeDtypeStruct((_B, _H, _W), jnp.float32),
            jax.ShapeDtypeStruct((_B, _H, _W), jnp.float32),
        ],
    )(score, target)


_NC, _NS, _L = 2, 16, 16
_NW = _NC * _NS
_HB = _B // 2
_HROWS = _HB * _H
_TROWS = _HROWS // _NW
_Q = _TROWS // 4
_VPR = _W // _L


def _sc_quarter(buf, accs):
    def row_body(r, carry):
        a = list(carry)
        for j in range(_VPR):
            v = buf[r, pl.ds(j * _L, _L)]
            k = j % 4
            a[k] = a[k] + jnp.where(v > _KEPT_MIN_LOSS, 1.0, 0.0)
            a[4 + k] = a[4 + k] + v
        return tuple(a)

    return lax.fori_loop(0, _Q, row_body, accs)


def _sc_body(ml_hbm, out_hbm, b0, b1, acc_v, s0, s1):
    wid = lax.axis_index("s") * _NC + lax.axis_index("c")
    r0 = wid * _TROWS
    bufs, sems = (b0, b1), (s0, s1)

    def fire(q, cur):
        rq = r0 + q * _Q
        return pltpu.async_copy(ml_hbm.at[pl.ds(rq, _Q)], bufs[cur], sems[cur])

    zero = jnp.zeros((_L,), jnp.float32)
    accs = tuple([zero] * 8)
    inflight = fire(0, 0)
    for q in range(4):
        cur = q % 2
        cp = inflight
        if q < 3:
            inflight = fire(q + 1, 1 - cur)
        cp.wait()
        accs = _sc_quarter(bufs[cur], accs)
    c06 = accs[0] + accs[1] + accs[2] + accs[3]
    s06 = accs[4] + accs[5] + accs[6] + accs[7]
    acc_v[pl.ds(0, _L)] = c06
    acc_v[pl.ds(_L, _L)] = s06
    pltpu.sync_copy(acc_v, out_hbm.at[wid])


_sc_pass2 = functools.partial(
    pl.kernel,
    out_type=jax.ShapeDtypeStruct((_NW, 2 * _L), jnp.float32),
    mesh=plsc.VectorSubcoreMesh(core_axis_name="c", subcore_axis_name="s"),
    compiler_params=pltpu.CompilerParams(use_tc_tiling_on_sc=True),
    scratch_types=[
        pltpu.VMEM((_Q, _W), jnp.float32),
        pltpu.VMEM((_Q, _W), jnp.float32),
        pltpu.VMEM((2 * _L,), jnp.float32),
        pltpu.SemaphoreType.DMA,
        pltpu.SemaphoreType.DMA,
    ],
)(_sc_body)


def _bisect_body(pred_ref, loss_ref, out_ref, thr_ref):
    p = pred_ref[...]
    l = loss_ref[...]
    bits = jax.lax.bitcast_convert_type(p, jnp.int32)
    n = jnp.sum(jnp.where(bits < _INF_BITS, 1.0, 0.0)).astype(jnp.int32)
    k = jnp.minimum(jnp.int32(_MIN_KEPT), n - 1)
    c06 = jnp.sum(jnp.where(bits < _THRESH_BITS, 1.0, 0.0)).astype(jnp.int32)
    thr_ref[0] = jnp.int32(_THRESH_BITS)

    @pl.when(c06 < k + 1)
    def _bisect_loop():
        def body(_, lohi):
            lo, hi = lohi
            mid = jax.lax.div(lo + hi, jnp.int32(2))
            cnt = jnp.sum(jnp.where(bits <= mid, 1.0, 0.0)).astype(jnp.int32)
            good = cnt >= k + 1
            return (jnp.where(good, lo, mid + 1), jnp.where(good, mid, hi))

        lo, _hi = jax.lax.fori_loop(
            0, 31, body, (jnp.int32(0), jnp.int32(0x40000000))
        )
        thr_ref[0] = jnp.maximum(lo, jnp.int32(_THRESH_BITS))

    thr = thr_ref[0]
    keep = bits < thr
    cnt = jnp.sum(jnp.where(keep, 1.0, 0.0))
    s = jnp.sum(jnp.where(keep, l, 0.0))
    out_ref[0] = s / jnp.maximum(cnt, 1.0)


def _bisect(pred2, loss2):
    rows, cols = pred2.shape
    return pl.pallas_call(
        _bisect_body,
        in_specs=[
            pl.BlockSpec((rows, cols), lambda: (0, 0)),
            pl.BlockSpec((rows, cols), lambda: (0, 0)),
        ],
        out_specs=pl.BlockSpec(memory_space=pltpu.SMEM),
        out_shape=jax.ShapeDtypeStruct((1,), jnp.float32),
        scratch_shapes=[pltpu.SMEM((1,), jnp.int32)],
    )(pred2, loss2)


def _rare_exact(score, target):
    loss_arr, pred_arr = _pass1_full(score, target)
    return _bisect(
        pred_arr.reshape(_ROWS, _W), loss_arr.reshape(_ROWS, _W)
    )[0]


def kernel(score, target):
    ml0 = _pass1(score, target, 0, _HB)
    ml1 = _pass1(score, target, _HB, _HB)
    p0 = _sc_pass2(ml0.reshape(_HROWS, _W))
    p1 = _sc_pass2(ml1.reshape(_HROWS, _W))
    partials = p0 + p1
    c06 = jnp.sum(partials[:, :_L])
    s06 = jnp.sum(partials[:, _L:])
    ohem = lax.cond(
        c06 >= jnp.float32(_MIN_KEPT + 1),
        lambda: s06 / c06,
        lambda: _rare_exact(score, target),
    )
    return (ohem, ohem, ohem - ohem)


# speedup vs baseline: 1.0571x; 1.0571x over previous
import functools

import jax
import jax.numpy as jnp
from jax import lax
from jax.experimental import pallas as pl
from jax.experimental.pallas import tpu as pltpu
from jax.experimental.pallas import tpu_sc as plsc

_IGNORE = 255
_THRESH = 0.6
_THRESH_BITS = 0x3F19999A
_INF_BITS = 0x7F800000
_MIN_KEPT = 256
_KEPT_MIN_LOSS = 0.25

_B, _C, _H, _W = 8, 19, 512, 512
_BH = 512
_N = _B * _H * _W
_ROWS = _N // _W


def _softmax_stats(score_ref, tgt_ref):
    t = tgt_ref[0]
    m = score_ref[0, 0]
    for c in range(1, _C):
        m = jnp.maximum(m, score_ref[0, c])
    s = jnp.zeros_like(m)
    xt = jnp.zeros_like(m)
    for c in range(_C):
        xc = score_ref[0, c]
        s = s + jnp.exp(xc - m)
        xt = jnp.where(t == c, xc, xt)
    logz = m + jnp.log(s)
    mask = t != _IGNORE
    return mask, logz, xt


def _pass1_body(score_ref, tgt_ref, ml_ref):
    mask, logz, xt = _softmax_stats(score_ref, tgt_ref)
    loss = logz - xt
    pred = jnp.exp(xt - logz)
    ml_ref[0] = jnp.where(mask & (pred < _THRESH), loss, 0.0)


def _pass1(score, target, b0, nb):
    nh = _H // _BH
    return pl.pallas_call(
        _pass1_body,
        grid=(nb, nh),
        in_specs=[
            pl.BlockSpec((1, _C, _BH, _W), lambda b, h: (b + b0, 0, h, 0)),
            pl.BlockSpec((1, _BH, _W), lambda b, h: (b + b0, h, 0)),
        ],
        out_specs=pl.BlockSpec((1, _BH, _W), lambda b, h: (b, h, 0)),
        out_shape=jax.ShapeDtypeStruct((nb, _H, _W), jnp.float32),
    )(score, target)


def _pass1_full_body(score_ref, tgt_ref, loss_ref, pred_ref):
    mask, logz, xt = _softmax_stats(score_ref, tgt_ref)
    loss_ref[0] = jnp.where(mask, logz - xt, 0.0)
    pred_ref[0] = jnp.where(mask, jnp.exp(xt - logz), jnp.inf)


def _pass1_full(score, target):
    nh = _H // _BH
    return pl.pallas_call(
        _pass1_full_body,
        grid=(_B, nh),
        in_specs=[
            pl.BlockSpec((1, _C, _BH, _W), lambda b, h: (b, 0, h, 0)),
            pl.BlockSpec((1, _BH, _W), lambda b, h: (b, h, 0)),
        ],
        out_specs=[
            pl.BlockSpec((1, _BH, _W), lambda b, h: (b, h, 0)),
            pl.BlockSpec((1, _BH, _W), lambda b, h: (b, h, 0)),
        ],
        out_shape=[
            jax.ShapeDtypeStruct((_B, _H, _W), jnp.float32),
            jax.ShapeDtypeStruct((_B, _H, _W), jnp.float32),
        ],
    )(score, target)


_NC, _NS, _L = 2, 16, 16
_NW = _NC * _NS
_TROWS = _ROWS // _NW
_Q = _TROWS // 4
_VPR = _W // _L


def _sc_quarter(buf, accs):
    def row_body(r, carry):
        a = list(carry)
        for j in range(_VPR):
            v = buf[r, pl.ds(j * _L, _L)]
            k = j % 4
            a[k] = a[k] + jnp.where(v > _KEPT_MIN_LOSS, 1.0, 0.0)
            a[4 + k] = a[4 + k] + v
        return tuple(a)

    return lax.fori_loop(0, _Q, row_body, accs)


def _sc_body(ml_hbm, out_hbm, b0, b1, acc_v, s0, s1):
    wid = lax.axis_index("s") * _NC + lax.axis_index("c")
    r0 = wid * _TROWS
    bufs, sems = (b0, b1), (s0, s1)

    def fire(q, cur):
        rq = r0 + q * _Q
        return pltpu.async_copy(ml_hbm.at[pl.ds(rq, _Q)], bufs[cur], sems[cur])

    zero = jnp.zeros((_L,), jnp.float32)
    accs = tuple([zero] * 8)
    inflight = fire(0, 0)
    for q in range(4):
        cur = q % 2
        cp = inflight
        if q < 3:
            inflight = fire(q + 1, 1 - cur)
        cp.wait()
        accs = _sc_quarter(bufs[cur], accs)
    c06 = accs[0] + accs[1] + accs[2] + accs[3]
    s06 = accs[4] + accs[5] + accs[6] + accs[7]
    acc_v[pl.ds(0, _L)] = c06
    acc_v[pl.ds(_L, _L)] = s06
    pltpu.sync_copy(acc_v, out_hbm.at[wid])


_sc_pass2 = functools.partial(
    pl.kernel,
    out_type=jax.ShapeDtypeStruct((_NW, 2 * _L), jnp.float32),
    mesh=plsc.VectorSubcoreMesh(core_axis_name="c", subcore_axis_name="s"),
    compiler_params=pltpu.CompilerParams(use_tc_tiling_on_sc=True),
    scratch_types=[
        pltpu.VMEM((_Q, _W), jnp.float32),
        pltpu.VMEM((_Q, _W), jnp.float32),
        pltpu.VMEM((2 * _L,), jnp.float32),
        pltpu.SemaphoreType.DMA,
        pltpu.SemaphoreType.DMA,
    ],
)(_sc_body)


def _bisect_body(pred_ref, loss_ref, out_ref, thr_ref):
    p = pred_ref[...]
    l = loss_ref[...]
    bits = jax.lax.bitcast_convert_type(p, jnp.int32)
    n = jnp.sum(jnp.where(bits < _INF_BITS, 1.0, 0.0)).astype(jnp.int32)
    k = jnp.minimum(jnp.int32(_MIN_KEPT), n - 1)
    c06 = jnp.sum(jnp.where(bits < _THRESH_BITS, 1.0, 0.0)).astype(jnp.int32)
    thr_ref[0] = jnp.int32(_THRESH_BITS)

    @pl.when(c06 < k + 1)
    def _bisect_loop():
        def body(_, lohi):
            lo, hi = lohi
            mid = jax.lax.div(lo + hi, jnp.int32(2))
            cnt = jnp.sum(jnp.where(bits <= mid, 1.0, 0.0)).astype(jnp.int32)
            good = cnt >= k + 1
            return (jnp.where(good, lo, mid + 1), jnp.where(good, mid, hi))

        lo, _hi = jax.lax.fori_loop(
            0, 31, body, (jnp.int32(0), jnp.int32(0x40000000))
        )
        thr_ref[0] = jnp.maximum(lo, jnp.int32(_THRESH_BITS))

    thr = thr_ref[0]
    keep = bits < thr
    cnt = jnp.sum(jnp.where(keep, 1.0, 0.0))
    s = jnp.sum(jnp.where(keep, l, 0.0))
    out_ref[0] = s / jnp.maximum(cnt, 1.0)


def _bisect(pred2, loss2):
    rows, cols = pred2.shape
    return pl.pallas_call(
        _bisect_body,
        in_specs=[
            pl.BlockSpec((rows, cols), lambda: (0, 0)),
            pl.BlockSpec((rows, cols), lambda: (0, 0)),
        ],
        out_specs=pl.BlockSpec(memory_space=pltpu.SMEM),
        out_shape=jax.ShapeDtypeStruct((1,), jnp.float32),
        scratch_shapes=[pltpu.SMEM((1,), jnp.int32)],
    )(pred2, loss2)


def _rare_exact(score, target):
    loss_arr, pred_arr = _pass1_full(score, target)
    return _bisect(
        pred_arr.reshape(_ROWS, _W), loss_arr.reshape(_ROWS, _W)
    )[0]


def kernel(score, target):
    ml = _pass1(score, target, 0, _B)
    partials = _sc_pass2(ml.reshape(_ROWS, _W))
    c06 = jnp.sum(partials[:, :_L])
    s06 = jnp.sum(partials[:, _L:])
    ohem = lax.cond(
        c06 >= jnp.float32(_MIN_KEPT + 1),
        lambda: s06 / c06,
        lambda: _rare_exact(score, target),
    )
    return (ohem, ohem, ohem - ohem)
